# baseline (speedup 1.0000x reference)
import functools

import jax
import jax.numpy as jnp
from jax import lax
from jax.experimental import pallas as pl
from jax.experimental.pallas import tpu as pltpu
from jax.experimental.pallas import tpu_sc as plsc

_BT = 512
_LOG_BT = 9
_NBPAD = 24


def _shift_cumsum(a, t):
    s = 1
    while s < t:
        pad = jnp.zeros((s, a.shape[1]), a.dtype)
        a = a + jnp.concatenate([pad, a[:-s, :]], axis=0)
        s *= 2
    return a


def _router_body(x_ref, g_ref, rl_ref, meta_ref, pw_ref, beact_ref,
                 pr0_ref, pr1_ref, *, n_e, t):
    xb = x_ref[...]
    rl = jnp.dot(xb, g_ref[...].T, preferred_element_type=jnp.float32)
    rl_ref[...] = rl

    idx = lax.broadcasted_iota(jnp.int32, rl.shape, 1)
    m1 = jnp.max(rl, axis=-1, keepdims=True)
    i1 = jnp.min(jnp.where(rl == m1, idx, n_e), axis=-1, keepdims=True)
    rl2 = jnp.where(idx == i1, -jnp.inf, rl)
    m2 = jnp.max(rl2, axis=-1, keepdims=True)
    i2 = jnp.min(jnp.where(rl2 == m2, idx, n_e), axis=-1, keepdims=True)
    z = jnp.exp(m2 - m1)
    p1 = 1.0 / (1.0 + z)
    p2 = z / (1.0 + z)

    oh0 = (idx == i1).astype(jnp.int32)
    oh1 = (idx == i2).astype(jnp.int32)
    c0 = _shift_cumsum(oh0, t)
    c1 = _shift_cumsum(oh1, t)
    r0 = jnp.sum(oh0 * c0, axis=1, keepdims=True) - 1
    t0 = c0[t - 1 : t, :]
    r1 = (jnp.sum(oh1 * c1, axis=1, keepdims=True) - 1
          + jnp.sum(oh1 * t0, axis=1, keepdims=True))
    total = t0 + c1[t - 1 : t, :]
    padded = ((total + (_BT - 1)) >> _LOG_BT) << _LOG_BT
    lt = (lax.broadcasted_iota(jnp.int32, (n_e, n_e), 0)
          <= lax.broadcasted_iota(jnp.int32, (n_e, n_e), 1)).astype(jnp.float32)
    cum = jnp.dot(padded.astype(jnp.float32), lt,
                  preferred_element_type=jnp.float32).astype(jnp.int32)
    base = cum - padded
    pos0 = jnp.sum(oh0 * base, axis=1, keepdims=True) + r0
    pos1 = jnp.sum(oh1 * base, axis=1, keepdims=True) + r1

    zi = jnp.zeros((t, n_e - 2), jnp.int32)
    zf = jnp.zeros((t, n_e - 2), jnp.float32)
    meta_ref[...] = jnp.concatenate([pos0, pos1, zi], axis=1)
    pw_ref[...] = jnp.concatenate([p1, p2, zf], axis=1)
    pr0_ref[...] = jnp.broadcast_to(p1, (t, 128))
    pr1_ref[...] = jnp.broadcast_to(p2, (t, 128))

    bs = lax.broadcasted_iota(jnp.int32, (_NBPAD, n_e), 0) * _BT
    ge = (bs >= cum).astype(jnp.int32)
    be = jnp.minimum(jnp.sum(ge, axis=1, keepdims=True), n_e - 1)
    act = (bs[:, 0:1] < cum[0:1, n_e - 1 : n_e]).astype(jnp.int32)
    zb = jnp.zeros((_NBPAD, n_e - 2), jnp.int32)
    beact_ref[...] = jnp.concatenate([be, act, zb], axis=1)


def _ffn_body(be_ref, act_ref, xs_ref, ws_ref, w1_ref, w2_ref, w3_ref, y_ref):
    b = pl.program_id(0)

    @pl.when(act_ref[b] == 1)
    def _():
        xb = xs_ref[...]
        h1 = jnp.dot(xb, w1_ref[0].T, preferred_element_type=jnp.float32)
        h3 = jnp.dot(xb, w3_ref[0].T, preferred_element_type=jnp.float32)
        h = h1 * lax.logistic(h1) * h3
        y = jnp.dot(h, w2_ref[0].T, preferred_element_type=jnp.float32)
        y_ref[...] = y * ws_ref[:, 0:1]


def _combine_body(y0_ref, y1_ref, pw_ref, out_ref):
    w0 = pw_ref[:, 0:1]
    w1 = pw_ref[:, 1:2]
    out_ref[...] = w0 * y0_ref[...] + w1 * y1_ref[...]


def kernel(x, gate_w, w1, w2, w3):
    B, L, D = x.shape
    E, F, _ = w1.shape
    T = B * L
    P = 2 * T + E * _BT
    NB = P // _BT
    xf = x.reshape(T, D)

    rl, meta, pw, beact, pr0, pr1 = pl.pallas_call(
        functools.partial(_router_body, n_e=E, t=T),
        out_shape=[
            jax.ShapeDtypeStruct((T, E), jnp.float32),
            jax.ShapeDtypeStruct((T, E), jnp.int32),
            jax.ShapeDtypeStruct((T, E), jnp.float32),
            jax.ShapeDtypeStruct((_NBPAD, E), jnp.int32),
            jax.ShapeDtypeStruct((T, 128), jnp.float32),
            jax.ShapeDtypeStruct((T, 128), jnp.float32),
        ],
    )(xf, gate_w)
    pos0 = meta[:, 0]
    pos1 = meta[:, 1]
    be = beact[:, 0]
    act = beact[:, 1]

    mesh = plsc.VectorSubcoreMesh(core_axis_name="c", subcore_axis_name="s")
    n_sub = mesh.num_cores * mesh.num_subcores
    NT = T // n_sub

    @functools.partial(
        pl.kernel,
        out_type=(
            jax.ShapeDtypeStruct((P, D), jnp.float32),
            jax.ShapeDtypeStruct((P, 128), jnp.float32),
        ),
        mesh=mesh,
        scratch_types=[
            pltpu.VMEM((NT,), jnp.int32),
            pltpu.VMEM((NT, D), jnp.float32),
            pltpu.VMEM((NT, 128), jnp.float32),
            pltpu.SemaphoreType.DMA,
        ],
    )
    def _scatter_k(x_hbm, p0_hbm, p1_hbm, w0_hbm, w1_hbm, xs_hbm, ws_hbm,
                   idx_v, rows_v, wrow_v, sem):
        wid = lax.axis_index("s") * mesh.num_cores + lax.axis_index("c")
        tb = wid * NT
        pltpu.sync_copy(x_hbm.at[pl.ds(tb, NT), :], rows_v)
        for p_hbm, w_hbm in ((p0_hbm, w0_hbm), (p1_hbm, w1_hbm)):
            pltpu.sync_copy(p_hbm.at[pl.ds(tb, NT)], idx_v)
            pltpu.async_copy(rows_v, xs_hbm.at[idx_v], sem).wait()
            pltpu.sync_copy(w_hbm.at[pl.ds(tb, NT), :], wrow_v)
            pltpu.async_copy(wrow_v, ws_hbm.at[idx_v], sem).wait()

    xs, ws = _scatter_k(xf, pos0, pos1, pr0, pr1)

    y = pl.pallas_call(
        _ffn_body,
        grid_spec=pltpu.PrefetchScalarGridSpec(
            num_scalar_prefetch=2,
            grid=(NB,),
            in_specs=[
                pl.BlockSpec((_BT, D), lambda b, be_r, act_r: (b, 0)),
                pl.BlockSpec((_BT, 128), lambda b, be_r, act_r: (b, 0)),
                pl.BlockSpec((1, F, D), lambda b, be_r, act_r: (be_r[b], 0, 0)),
                pl.BlockSpec((1, D, F), lambda b, be_r, act_r: (be_r[b], 0, 0)),
                pl.BlockSpec((1, F, D), lambda b, be_r, act_r: (be_r[b], 0, 0)),
            ],
            out_specs=pl.BlockSpec((_BT, D), lambda b, be_r, act_r: (b, 0)),
        ),
        out_shape=jax.ShapeDtypeStruct((P, D), jnp.float32),
        compiler_params=pltpu.CompilerParams(
            vmem_limit_bytes=100 * 1024 * 1024),
    )(be, act, xs, ws, w1, w2, w3)

    NH = NT // 2

    @functools.partial(
        pl.kernel,
        out_type=jax.ShapeDtypeStruct((T, D), jnp.float32),
        mesh=mesh,
        scratch_types=[
            pltpu.VMEM((NH,), jnp.int32),
            pltpu.VMEM((NH, D), jnp.float32),
            pltpu.VMEM((NH, D), jnp.float32),
            pltpu.SemaphoreType.DMA,
        ],
    )
    def _gather_k(y_hbm, p0_hbm, p1_hbm, out_hbm, idx_v, r0_v, r1_v, sem):
        wid = lax.axis_index("s") * mesh.num_cores + lax.axis_index("c")
        for h in range(2):
            tb = wid * NT + h * NH
            pltpu.sync_copy(p0_hbm.at[pl.ds(tb, NH)], idx_v)
            pltpu.async_copy(y_hbm.at[idx_v], r0_v, sem).wait()
            pltpu.sync_copy(p1_hbm.at[pl.ds(tb, NH)], idx_v)
            pltpu.async_copy(y_hbm.at[idx_v], r1_v, sem).wait()

            def _add_row(i, _):
                for j in range(D // 16):
                    sl = pl.ds(j * 16, 16)
                    r0_v[i, sl] = r0_v[i, sl] + r1_v[i, sl]
                return 0

            lax.fori_loop(0, NH, _add_row, 0)
            pltpu.sync_copy(r0_v, out_hbm.at[pl.ds(tb, NH), :])

    out = _gather_k(y, pos0, pos1)

    return out.reshape(B, L, D), rl


# speedup vs baseline: 1.0019x; 1.0019x over previous
import functools

import jax
import jax.numpy as jnp
from jax import lax
from jax.experimental import pallas as pl
from jax.experimental.pallas import tpu as pltpu
from jax.experimental.pallas import tpu_sc as plsc

_BT = 512
_LOG_BT = 9
_NBPAD = 24


def _shift_cumsum(a, t):
    s = 1
    while s < t:
        pad = jnp.zeros((s, a.shape[1]), a.dtype)
        a = a + jnp.concatenate([pad, a[:-s, :]], axis=0)
        s *= 2
    return a


def _router_body(x_ref, g_ref, rl_ref, meta_ref, pw_ref, beact_ref, *, n_e, t):
    xb = x_ref[...]
    rl = jnp.dot(xb, g_ref[...].T, preferred_element_type=jnp.float32)
    rl_ref[...] = rl

    idx = lax.broadcasted_iota(jnp.int32, rl.shape, 1)
    m1 = jnp.max(rl, axis=-1, keepdims=True)
    i1 = jnp.min(jnp.where(rl == m1, idx, n_e), axis=-1, keepdims=True)
    rl2 = jnp.where(idx == i1, -jnp.inf, rl)
    m2 = jnp.max(rl2, axis=-1, keepdims=True)
    i2 = jnp.min(jnp.where(rl2 == m2, idx, n_e), axis=-1, keepdims=True)
    z = jnp.exp(m2 - m1)
    p1 = 1.0 / (1.0 + z)
    p2 = z / (1.0 + z)

    oh0 = (idx == i1).astype(jnp.int32)
    oh1 = (idx == i2).astype(jnp.int32)
    c0 = _shift_cumsum(oh0, t)
    c1 = _shift_cumsum(oh1, t)
    r0 = jnp.sum(oh0 * c0, axis=1, keepdims=True) - 1
    t0 = c0[t - 1 : t, :]
    r1 = (jnp.sum(oh1 * c1, axis=1, keepdims=True) - 1
          + jnp.sum(oh1 * t0, axis=1, keepdims=True))
    total = t0 + c1[t - 1 : t, :]
    padded = ((total + (_BT - 1)) >> _LOG_BT) << _LOG_BT
    lt = (lax.broadcasted_iota(jnp.int32, (n_e, n_e), 0)
          <= lax.broadcasted_iota(jnp.int32, (n_e, n_e), 1)).astype(jnp.float32)
    cum = jnp.dot(padded.astype(jnp.float32), lt,
                  preferred_element_type=jnp.float32).astype(jnp.int32)
    base = cum - padded
    pos0 = jnp.sum(oh0 * base, axis=1, keepdims=True) + r0
    pos1 = jnp.sum(oh1 * base, axis=1, keepdims=True) + r1

    zi = jnp.zeros((t, n_e - 2), jnp.int32)
    zf = jnp.zeros((t, n_e - 2), jnp.float32)
    meta_ref[...] = jnp.concatenate([pos0, pos1, zi], axis=1)
    pw_ref[...] = jnp.concatenate([p1, p2, zf], axis=1)

    bs = lax.broadcasted_iota(jnp.int32, (_NBPAD, n_e), 0) * _BT
    ge = (bs >= cum).astype(jnp.int32)
    be = jnp.minimum(jnp.sum(ge, axis=1, keepdims=True), n_e - 1)
    act = (bs[:, 0:1] < cum[0:1, n_e - 1 : n_e]).astype(jnp.int32)
    zb = jnp.zeros((_NBPAD, n_e - 2), jnp.int32)
    beact_ref[...] = jnp.concatenate([be, act, zb], axis=1)


def _ffn_body(be_ref, act_ref, xs_ref, w1_ref, w2_ref, w3_ref, y_ref):
    b = pl.program_id(0)

    @pl.when(act_ref[b] == 1)
    def _():
        xb = xs_ref[...]
        h1 = jnp.dot(xb, w1_ref[0].T, preferred_element_type=jnp.float32)
        h3 = jnp.dot(xb, w3_ref[0].T, preferred_element_type=jnp.float32)
        h = h1 * lax.logistic(h1) * h3
        y_ref[...] = jnp.dot(h, w2_ref[0].T, preferred_element_type=jnp.float32)


def _combine_body(y0_ref, y1_ref, pw_ref, out_ref):
    w0 = pw_ref[:, 0:1]
    w1 = pw_ref[:, 1:2]
    out_ref[...] = w0 * y0_ref[...] + w1 * y1_ref[...]


def kernel(x, gate_w, w1, w2, w3):
    B, L, D = x.shape
    E, F, _ = w1.shape
    T = B * L
    P = 2 * T + E * _BT
    NB = P // _BT
    xf = x.reshape(T, D)

    rl, meta, pw, beact = pl.pallas_call(
        functools.partial(_router_body, n_e=E, t=T),
        out_shape=[
            jax.ShapeDtypeStruct((T, E), jnp.float32),
            jax.ShapeDtypeStruct((T, E), jnp.int32),
            jax.ShapeDtypeStruct((T, E), jnp.float32),
            jax.ShapeDtypeStruct((_NBPAD, E), jnp.int32),
        ],
    )(xf, gate_w)
    pos0 = meta[:, 0]
    pos1 = meta[:, 1]
    be = beact[:, 0]
    act = beact[:, 1]

    mesh = plsc.VectorSubcoreMesh(core_axis_name="c", subcore_axis_name="s")
    n_sub = mesh.num_cores * mesh.num_subcores
    NT = T // n_sub

    @functools.partial(
        pl.kernel,
        out_type=jax.ShapeDtypeStruct((P, D), jnp.float32),
        mesh=mesh,
        scratch_types=[
            pltpu.VMEM((NT,), jnp.int32),
            pltpu.VMEM((NT, D), jnp.float32),
            pltpu.SemaphoreType.DMA,
        ],
    )
    def _scatter_k(x_hbm, p0_hbm, p1_hbm, xs_hbm, idx_v, rows_v, sem):
        wid = lax.axis_index("s") * mesh.num_cores + lax.axis_index("c")
        tb = wid * NT
        pltpu.sync_copy(x_hbm.at[pl.ds(tb, NT), :], rows_v)
        pltpu.sync_copy(p0_hbm.at[pl.ds(tb, NT)], idx_v)
        pltpu.async_copy(rows_v, xs_hbm.at[idx_v], sem).wait()
        pltpu.sync_copy(p1_hbm.at[pl.ds(tb, NT)], idx_v)
        pltpu.async_copy(rows_v, xs_hbm.at[idx_v], sem).wait()

    xs = _scatter_k(xf, pos0, pos1)

    y = pl.pallas_call(
        _ffn_body,
        grid_spec=pltpu.PrefetchScalarGridSpec(
            num_scalar_prefetch=2,
            grid=(NB,),
            in_specs=[
                pl.BlockSpec((_BT, D), lambda b, be_r, act_r: (b, 0)),
                pl.BlockSpec((1, F, D), lambda b, be_r, act_r: (be_r[b], 0, 0)),
                pl.BlockSpec((1, D, F), lambda b, be_r, act_r: (be_r[b], 0, 0)),
                pl.BlockSpec((1, F, D), lambda b, be_r, act_r: (be_r[b], 0, 0)),
            ],
            out_specs=pl.BlockSpec((_BT, D), lambda b, be_r, act_r: (b, 0)),
        ),
        out_shape=jax.ShapeDtypeStruct((P, D), jnp.float32),
        compiler_params=pltpu.CompilerParams(
            vmem_limit_bytes=100 * 1024 * 1024),
    )(be, act, xs, w1, w2, w3)

    @functools.partial(
        pl.kernel,
        out_type=(
            jax.ShapeDtypeStruct((T, D), jnp.float32),
            jax.ShapeDtypeStruct((T, D), jnp.float32),
        ),
        mesh=mesh,
        scratch_types=[
            pltpu.VMEM((NT,), jnp.int32),
            pltpu.VMEM((NT, D), jnp.float32),
            pltpu.SemaphoreType.DMA,
        ],
    )
    def _gather_k(y_hbm, p0_hbm, p1_hbm, y0_hbm, y1_hbm, idx_v, rows_v, sem):
        wid = lax.axis_index("s") * mesh.num_cores + lax.axis_index("c")
        tb = wid * NT
        pltpu.sync_copy(p0_hbm.at[pl.ds(tb, NT)], idx_v)
        pltpu.async_copy(y_hbm.at[idx_v], rows_v, sem).wait()
        pltpu.sync_copy(rows_v, y0_hbm.at[pl.ds(tb, NT), :])
        pltpu.sync_copy(p1_hbm.at[pl.ds(tb, NT)], idx_v)
        pltpu.async_copy(y_hbm.at[idx_v], rows_v, sem).wait()
        pltpu.sync_copy(rows_v, y1_hbm.at[pl.ds(tb, NT), :])

    y0g, y1g = _gather_k(y, pos0, pos1)

    BT2 = 256
    out = pl.pallas_call(
        _combine_body,
        grid=(T // BT2,),
        in_specs=[
            pl.BlockSpec((BT2, D), lambda b: (b, 0)),
            pl.BlockSpec((BT2, D), lambda b: (b, 0)),
            pl.BlockSpec((BT2, E), lambda b: (b, 0)),
        ],
        out_specs=pl.BlockSpec((BT2, D), lambda b: (b, 0)),
        out_shape=jax.ShapeDtypeStruct((T, D), jnp.float32),
    )(y0g, y1g, pw)

    return out.reshape(B, L, D), rl
